# baseline (speedup 1.0000x reference)
import functools

import jax
import jax.numpy as jnp
import numpy as np
from jax import lax
from jax.experimental import pallas as pl
from jax.experimental.pallas import tpu as pltpu
from jax.experimental.pallas import tpu_sc as plsc

D_MODEL, N_HEADS, N_LEVELS, N_POINTS, HEAD_DIM = 256, 8, 4, 4, 32
_SPATIAL = [(64, 64), (32, 32), (16, 16), (8, 8)]
_SIZES = [h * w for h, w in _SPATIAL]
_STARTS = np.cumsum([0] + _SIZES)
_B, _LQ = 2, int(sum(_SIZES))
_NROW = _B * _LQ
_NP_L = [(h + 2) * (w + 2) for h, w in _SPATIAL]
_PSTART = np.cumsum([0] + _NP_L)
_PTOT = int(_PSTART[-1])

_lane = np.arange(128)
_L_OF = (_lane // 4) % 4
_W_LANE = np.array([_SPATIAL[l][1] for l in _L_OF], np.float32)
_H_LANE = np.array([_SPATIAL[l][0] for l in _L_OF], np.float32)
_WI_LANE = _W_LANE.astype(np.int32)
_HI_LANE = _H_LANE.astype(np.int32)
_S_LANE = np.array([_PSTART[l] for l in _L_OF], np.int32)
_PWR_LANE = np.array([_SPATIAL[l][1] + 2 for l in _L_OF], np.int32)

_RBLK = 680
_NSTEP = _NROW // _RBLK


def _proj_body(q_ref, x_ref, wval_ref, bval_ref, wox_ref, box_ref, woy_ref,
               boy_ref, wat_ref, bat_ref, rx_ref, ry_ref,
               wl_ref, hl_ref, wi_ref, hi_ref, s_ref, pwr_ref,
               val_ref, idx_ref, w4_ref):
    f32 = jnp.float32
    bf16 = jnp.bfloat16
    q = q_ref[...].astype(bf16)
    val_ref[...] = (jnp.dot(x_ref[...].astype(bf16), wval_ref[...],
                            preferred_element_type=f32) + bval_ref[...])
    offx = jnp.dot(q, wox_ref[...], preferred_element_type=f32) + box_ref[...]
    offy = jnp.dot(q, woy_ref[...], preferred_element_type=f32) + boy_ref[...]
    logits = jnp.dot(q, wat_ref[...], preferred_element_type=f32) + bat_ref[...]
    m = jnp.max(logits, axis=-1, keepdims=True)
    e = jnp.exp(logits - m)
    li = lax.broadcasted_iota(jnp.int32, (128, 128), 0) // 16
    lj = lax.broadcasted_iota(jnp.int32, (128, 128), 1) // 16
    gmask = (li == lj).astype(bf16)
    aw = e / jnp.dot(e.astype(bf16), gmask, preferred_element_type=f32)

    wl = wl_ref[...]
    hl = hl_ref[...]
    wi = wi_ref[...]
    hi = hi_ref[...]
    gx = rx_ref[...] * wl + offx - 0.5
    gy = ry_ref[...] * hl + offy - 0.5
    x0 = jnp.floor(gx)
    y0 = jnp.floor(gy)
    x0i = x0.astype(jnp.int32)
    y0i = y0.astype(jnp.int32)
    wx1 = gx - x0
    wx0 = 1.0 - wx1
    wy1 = gy - y0
    wy0 = 1.0 - wy1
    okx = ((x0i >= -1) & (x0i <= wi - 1)).astype(f32)
    oky = ((y0i >= -1) & (y0i <= hi - 1)).astype(f32)
    cx = jnp.clip(x0i + 1, 0, wi)
    cy = jnp.clip(y0i + 1, 0, hi)
    p = cy * pwr_ref[...] + cx + s_ref[...]
    rows = (pl.program_id(0) * _RBLK
            + lax.broadcasted_iota(jnp.int32, (_RBLK, 128), 0))
    bsel = jnp.where(rows >= _LQ, 1, 0)
    hlane = lax.broadcasted_iota(jnp.int32, (_RBLK, 128), 1) // 16
    idx_ref[...] = (bsel * _PTOT + p) * 8 + hlane
    a = aw * okx * oky
    def _pack(lo, hi):
        lo16 = lax.bitcast_convert_type(lo.astype(bf16), jnp.uint16)
        hi16 = lax.bitcast_convert_type(hi.astype(bf16), jnp.uint16)
        word = lo16.astype(jnp.uint32) | (hi16.astype(jnp.uint32) << 16)
        return lax.bitcast_convert_type(word, jnp.int32)
    w4_ref[:, 0:128] = _pack(a * wx0 * wy0, a * wx1 * wy0)
    w4_ref[:, 128:256] = _pack(a * wx0 * wy1, a * wx1 * wy1)


def _run_proj(q2, x2, wvalT, bval, woxT, box, woyT, boy, watT, bat, rx, ry):
    f32 = jnp.float32
    blk = lambda c: pl.BlockSpec((_RBLK, c), lambda i: (i, 0))
    full = lambda r, c: pl.BlockSpec((r, c), lambda i: (0, 0))
    return pl.pallas_call(
        _proj_body,
        grid=(_NSTEP,),
        in_specs=[blk(256), blk(256), full(256, 256), full(1, 256),
                  full(256, 128), full(1, 128), full(256, 128), full(1, 128),
                  full(256, 128), full(1, 128), blk(128), blk(128)]
                 + [full(1, 128)] * 6,
        out_specs=[blk(256), blk(128), blk(256)],
        out_shape=[jax.ShapeDtypeStruct((_NROW, 256), f32),
                   jax.ShapeDtypeStruct((_NROW, 128), jnp.int32),
                   jax.ShapeDtypeStruct((_NROW, 256), jnp.int32)],
    )(q2, x2, wvalT, bval, woxT, box, woyT, boy, watT, bat, rx, ry,
      jnp.asarray(_W_LANE[None, :]), jnp.asarray(_H_LANE[None, :]),
      jnp.asarray(_WI_LANE[None, :]), jnp.asarray(_HI_LANE[None, :]),
      jnp.asarray(_S_LANE[None, :]), jnp.asarray(_PWR_LANE[None, :]))


def _matmul_body(x_ref, w_ref, b_ref, o_ref):
    o_ref[...] = (jnp.dot(x_ref[...].astype(jnp.bfloat16), w_ref[...],
                          preferred_element_type=jnp.float32) + b_ref[...])


def _run_matmul(x2, wT, b):
    return pl.pallas_call(
        _matmul_body,
        grid=(_NSTEP,),
        in_specs=[pl.BlockSpec((_RBLK, 256), lambda i: (i, 0)),
                  pl.BlockSpec((256, 256), lambda i: (0, 0)),
                  pl.BlockSpec((1, 256), lambda i: (0, 0))],
        out_specs=pl.BlockSpec((_RBLK, 256), lambda i: (i, 0)),
        out_shape=jax.ShapeDtypeStruct((_NROW, 256), jnp.float32),
    )(x2, wT, b)


def _build_table(value):
    v = value.reshape(_B, _LQ, 8, 32)
    tabs = []
    for lid, (h, w) in enumerate(_SPATIAL):
        s = int(_STARTS[lid])
        f = v[:, s:s + h * w].reshape(_B, h, w, 8, 32)
        f = jnp.pad(f, ((0, 0), (1, 1), (1, 1), (0, 0), (0, 0)))
        f = f.reshape(_B, _NP_L[lid], 8, 32)
        pw = w + 2
        tabs.append(jnp.stack(
            [f, jnp.roll(f, -1, axis=1), jnp.roll(f, -pw, axis=1),
             jnp.roll(f, -(pw + 1), axis=1)], axis=3))
    return jnp.concatenate(tabs, axis=1).reshape(_B * _PTOT * 8, 128)


_NW = 32
_ROWS_W = _NROW // _NW
_NS = 4
_NJ = _ROWS_W // _NS
_FLJ = 5
_FLROWS = _FLJ * _NS


def _sc_sample(table, idx_flat, w_flat):
    mesh = plsc.VectorSubcoreMesh(core_axis_name="c", subcore_axis_name="s")
    f32 = jnp.float32

    @functools.partial(
        pl.kernel, mesh=mesh,
        compiler_params=pltpu.CompilerParams(needs_layout_passes=False),
        out_type=jax.ShapeDtypeStruct((_NROW * 256,), f32),
        scratch_types=(
            [pltpu.VMEM((128,), jnp.int32)] * _NS
            + [pltpu.VMEM((256,), jnp.int32)] * _NS
            + [pltpu.VMEM((128, 128), f32)] * _NS
            + [pltpu.VMEM((_FLROWS * 256,), f32)]
            + [pltpu.SemaphoreType.DMA] * (2 * _NS)
        ),
    )
    def k(table_hbm, idx_hbm, w_hbm, out_hbm, *scr):
        idx_vs = scr[0:_NS]
        w_vs = scr[_NS:2 * _NS]
        rows_vs = scr[2 * _NS:3 * _NS]
        out_v = scr[3 * _NS]
        sem_iw = scr[3 * _NS + 1:3 * _NS + 1 + _NS]
        sem_g = scr[3 * _NS + 1 + _NS:]
        wid = lax.axis_index("s") * 2 + lax.axis_index("c")
        base = wid * _ROWS_W

        def iw_issue(blk, s):
            roff = base + blk
            pltpu.async_copy(idx_hbm.at[pl.ds(roff * 128, 128)], idx_vs[s],
                             sem_iw[s])
            pltpu.async_copy(w_hbm.at[pl.ds(roff * 256, 256)], w_vs[s],
                             sem_iw[s])

        def iw_wait(blk, s):
            roff = base + blk
            pltpu.make_async_copy(idx_hbm.at[pl.ds(roff * 128, 128)],
                                  idx_vs[s], sem_iw[s]).wait()
            pltpu.make_async_copy(w_hbm.at[pl.ds(roff * 256, 256)],
                                  w_vs[s], sem_iw[s]).wait()

        def g_issue(s):
            pltpu.async_copy(table_hbm.at[idx_vs[s]], rows_vs[s], sem_g[s])

        def g_wait(s):
            pltpu.make_async_copy(table_hbm.at[idx_vs[s]], rows_vs[s],
                                  sem_g[s]).wait()

        def compute(s, oloc):
            r = rows_vs[s]
            wv = w_vs[s]

            @pl.loop(0, 8)
            def _(h):
                acc0 = jnp.zeros((16,), f32)
                acc1 = jnp.zeros((16,), f32)
                for pt in range(16):
                    li = h * 16 + pt
                    s0 = jnp.full((16,), li, jnp.int32)
                    wab = plsc.bitcast(plsc.load_gather(wv, [s0]),
                                       jnp.bfloat16)
                    wcd = plsc.bitcast(plsc.load_gather(wv, [s0 + 128]),
                                       jnp.bfloat16)
                    v00, v01 = plsc.unpack(wab,
                                           format=plsc.PackFormat.INTERLEAVED)
                    v10, v11 = plsc.unpack(wcd,
                                           format=plsc.PackFormat.INTERLEAVED)
                    acc0 = (acc0 + v00 * r[li, pl.ds(0, 16)]
                            + v01 * r[li, pl.ds(32, 16)]
                            + v10 * r[li, pl.ds(64, 16)]
                            + v11 * r[li, pl.ds(96, 16)])
                    acc1 = (acc1 + v00 * r[li, pl.ds(16, 16)]
                            + v01 * r[li, pl.ds(48, 16)]
                            + v10 * r[li, pl.ds(80, 16)]
                            + v11 * r[li, pl.ds(112, 16)])
                out_v[pl.ds(oloc + h * 32, 16)] = acc0
                out_v[pl.ds(oloc + h * 32 + 16, 16)] = acc1

        for s in range(_NS):
            iw_issue(s, s)
        for s in range(2):
            iw_wait(s, s)
            g_issue(s)

        @pl.loop(0, _NJ)
        def _(j):
            local = j - (j // _FLJ) * _FLJ
            oloc = local * (_NS * 256)
            for s in range(_NS):
                blk = _NS * j + s
                g_wait(s)
                compute(s, oloc + s * 256)

                @pl.when(blk + _NS < _ROWS_W)
                def _():
                    iw_issue(blk + _NS, s)

                nblk = _NS * j + s + 2
                ns = (s + 2) % _NS

                @pl.when(nblk < _ROWS_W)
                def _():
                    iw_wait(nblk, ns)
                    g_issue(ns)

            @pl.when(local == _FLJ - 1)
            def _():
                fl = j // _FLJ
                pltpu.sync_copy(
                    out_v,
                    out_hbm.at[pl.ds(base * 256 + fl * (_FLROWS * 256),
                                     _FLROWS * 256)])

    return k(table, idx_flat, w_flat)


def kernel(query, reference_points, input_flatten, input_spatial_shapes,
           input_level_start_index, W_off, b_off, W_attn, b_attn, W_val,
           b_val, W_out, b_out):
    f32 = jnp.float32
    q2 = query.reshape(_NROW, 256)
    x2 = input_flatten.reshape(_NROW, 256)
    rp = reference_points.reshape(_NROW, 4, 2)
    rx = rp[:, _L_OF, 0]
    ry = rp[:, _L_OF, 1]
    bf16 = jnp.bfloat16
    val, idx, w4 = _run_proj(
        q2, x2,
        W_val.T.astype(bf16), b_val.reshape(1, 256),
        W_off[0::2].T.astype(bf16), b_off[0::2].reshape(1, 128),
        W_off[1::2].T.astype(bf16), b_off[1::2].reshape(1, 128),
        W_attn.T.astype(bf16), b_attn.reshape(1, 128),
        rx, ry)
    table = _build_table(val)
    acc = _sc_sample(table, idx.reshape(-1), w4.reshape(-1))
    out = _run_matmul(acc.reshape(_NROW, 256), W_out.T.astype(bf16),
                      b_out.reshape(1, 256))
    return out.reshape(_B, _LQ, 256)


# speedup vs baseline: 1.2281x; 1.2281x over previous
import functools

import jax
import jax.numpy as jnp
import numpy as np
from jax import lax
from jax.experimental import pallas as pl
from jax.experimental.pallas import tpu as pltpu
from jax.experimental.pallas import tpu_sc as plsc

D_MODEL, N_HEADS, N_LEVELS, N_POINTS, HEAD_DIM = 256, 8, 4, 4, 32
_SPATIAL = [(64, 64), (32, 32), (16, 16), (8, 8)]
_SIZES = [h * w for h, w in _SPATIAL]
_STARTS = np.cumsum([0] + _SIZES)
_B, _LQ = 2, int(sum(_SIZES))
_NROW = _B * _LQ
_NP_L = [(h + 2) * (w + 2) for h, w in _SPATIAL]
_PSTART = np.cumsum([0] + _NP_L)
_PTOT = int(_PSTART[-1])

_lane = np.arange(128)
_L_OF = (_lane // 4) % 4
_W_LANE = np.array([_SPATIAL[l][1] for l in _L_OF], np.float32)
_H_LANE = np.array([_SPATIAL[l][0] for l in _L_OF], np.float32)
_WI_LANE = _W_LANE.astype(np.int32)
_HI_LANE = _H_LANE.astype(np.int32)
_S_LANE = np.array([_PSTART[l] for l in _L_OF], np.int32)
_PWR_LANE = np.array([_SPATIAL[l][1] + 2 for l in _L_OF], np.int32)

_RBLK = 680
_NSTEP = _NROW // _RBLK


def _proj_body(q_ref, x_ref, wval_ref, bval_ref, wox_ref, box_ref, woy_ref,
               boy_ref, wat_ref, bat_ref, rx_ref, ry_ref,
               wl_ref, hl_ref, wi_ref, hi_ref, s_ref, pwr_ref,
               val_ref, idx_ref, w4_ref):
    f32 = jnp.float32
    bf16 = jnp.bfloat16
    q = q_ref[...].astype(bf16)
    val_ref[...] = (jnp.dot(x_ref[...].astype(bf16), wval_ref[...],
                            preferred_element_type=f32) + bval_ref[...])
    offx = jnp.dot(q, wox_ref[...], preferred_element_type=f32) + box_ref[...]
    offy = jnp.dot(q, woy_ref[...], preferred_element_type=f32) + boy_ref[...]
    logits = jnp.dot(q, wat_ref[...], preferred_element_type=f32) + bat_ref[...]
    m = jnp.max(logits, axis=-1, keepdims=True)
    e = jnp.exp(logits - m)
    li = lax.broadcasted_iota(jnp.int32, (128, 128), 0) // 16
    lj = lax.broadcasted_iota(jnp.int32, (128, 128), 1) // 16
    gmask = (li == lj).astype(bf16)
    aw = e / jnp.dot(e.astype(bf16), gmask, preferred_element_type=f32)

    wl = wl_ref[...]
    hl = hl_ref[...]
    wi = wi_ref[...]
    hi = hi_ref[...]
    gx = rx_ref[...] * wl + offx - 0.5
    gy = ry_ref[...] * hl + offy - 0.5
    x0 = jnp.floor(gx)
    y0 = jnp.floor(gy)
    x0i = x0.astype(jnp.int32)
    y0i = y0.astype(jnp.int32)
    wx1 = gx - x0
    wx0 = 1.0 - wx1
    wy1 = gy - y0
    wy0 = 1.0 - wy1
    okx = ((x0i >= -1) & (x0i <= wi - 1)).astype(f32)
    oky = ((y0i >= -1) & (y0i <= hi - 1)).astype(f32)
    cx = jnp.clip(x0i + 1, 0, wi)
    cy = jnp.clip(y0i + 1, 0, hi)
    p = cy * pwr_ref[...] + cx + s_ref[...]
    rows = (pl.program_id(0) * _RBLK
            + lax.broadcasted_iota(jnp.int32, (_RBLK, 128), 0))
    bsel = jnp.where(rows >= _LQ, 1, 0)
    hlane = lax.broadcasted_iota(jnp.int32, (_RBLK, 128), 1) // 16
    idx_ref[...] = (bsel * _PTOT + p) * 8 + hlane
    a = aw * okx * oky
    def _pack(lo, hi):
        lo16 = lax.bitcast_convert_type(lo.astype(bf16), jnp.uint16)
        hi16 = lax.bitcast_convert_type(hi.astype(bf16), jnp.uint16)
        word = lo16.astype(jnp.uint32) | (hi16.astype(jnp.uint32) << 16)
        return lax.bitcast_convert_type(word, jnp.int32)
    w4_ref[:, 0:128] = _pack(a * wx0 * wy0, a * wx1 * wy0)
    w4_ref[:, 128:256] = _pack(a * wx0 * wy1, a * wx1 * wy1)


def _run_proj(q2, x2, wvalT, bval, woxT, box, woyT, boy, watT, bat, rx, ry):
    f32 = jnp.float32
    blk = lambda c: pl.BlockSpec((_RBLK, c), lambda i: (i, 0))
    full = lambda r, c: pl.BlockSpec((r, c), lambda i: (0, 0))
    return pl.pallas_call(
        _proj_body,
        grid=(_NSTEP,),
        in_specs=[blk(256), blk(256), full(256, 256), full(1, 256),
                  full(256, 128), full(1, 128), full(256, 128), full(1, 128),
                  full(256, 128), full(1, 128), blk(128), blk(128)]
                 + [full(1, 128)] * 6,
        out_specs=[blk(256), blk(128), blk(256)],
        out_shape=[jax.ShapeDtypeStruct((_NROW, 256), f32),
                   jax.ShapeDtypeStruct((_NROW, 128), jnp.int32),
                   jax.ShapeDtypeStruct((_NROW, 256), jnp.int32)],
    )(q2, x2, wvalT, bval, woxT, box, woyT, boy, watT, bat, rx, ry,
      jnp.asarray(_W_LANE[None, :]), jnp.asarray(_H_LANE[None, :]),
      jnp.asarray(_WI_LANE[None, :]), jnp.asarray(_HI_LANE[None, :]),
      jnp.asarray(_S_LANE[None, :]), jnp.asarray(_PWR_LANE[None, :]))


def _matmul_body(x_ref, w_ref, b_ref, o_ref):
    o_ref[...] = (jnp.dot(x_ref[...].astype(jnp.bfloat16), w_ref[...],
                          preferred_element_type=jnp.float32) + b_ref[...])


def _run_matmul(x2, wT, b):
    return pl.pallas_call(
        _matmul_body,
        grid=(_NSTEP,),
        in_specs=[pl.BlockSpec((_RBLK, 256), lambda i: (i, 0)),
                  pl.BlockSpec((256, 256), lambda i: (0, 0)),
                  pl.BlockSpec((1, 256), lambda i: (0, 0))],
        out_specs=pl.BlockSpec((_RBLK, 256), lambda i: (i, 0)),
        out_shape=jax.ShapeDtypeStruct((_NROW, 256), jnp.float32),
    )(x2, wT, b)


def _build_table(value):
    v = value.reshape(_B, _LQ, 8, 32)
    tabs = []
    for lid, (h, w) in enumerate(_SPATIAL):
        s = int(_STARTS[lid])
        f = v[:, s:s + h * w].reshape(_B, h, w, 8, 32)
        f = jnp.pad(f, ((0, 0), (1, 1), (1, 1), (0, 0), (0, 0)))
        f = f.reshape(_B, _NP_L[lid], 8, 32)
        pw = w + 2
        tabs.append(jnp.concatenate(
            [f, jnp.roll(f, -1, axis=1), jnp.roll(f, -pw, axis=1),
             jnp.roll(f, -(pw + 1), axis=1)], axis=-1))
    return jnp.concatenate(tabs, axis=1).reshape(_B * _PTOT * 8, 128)


_NW = 32
_ROWS_W = _NROW // _NW
_NS = 4
_NJ = _ROWS_W // _NS
_FLJ = 5
_FLROWS = _FLJ * _NS


def _sc_sample(table, idx_flat, w_flat):
    mesh = plsc.VectorSubcoreMesh(core_axis_name="c", subcore_axis_name="s")
    f32 = jnp.float32

    @functools.partial(
        pl.kernel, mesh=mesh,
        compiler_params=pltpu.CompilerParams(needs_layout_passes=False),
        out_type=jax.ShapeDtypeStruct((_NROW * 256,), f32),
        scratch_types=(
            [pltpu.VMEM((128,), jnp.int32)] * _NS
            + [pltpu.VMEM((256,), jnp.int32)] * _NS
            + [pltpu.VMEM((128, 128), f32)] * _NS
            + [pltpu.VMEM((_FLROWS * 256,), f32)]
            + [pltpu.SemaphoreType.DMA] * (2 * _NS)
        ),
    )
    def k(table_hbm, idx_hbm, w_hbm, out_hbm, *scr):
        idx_vs = scr[0:_NS]
        w_vs = scr[_NS:2 * _NS]
        rows_vs = scr[2 * _NS:3 * _NS]
        out_v = scr[3 * _NS]
        sem_iw = scr[3 * _NS + 1:3 * _NS + 1 + _NS]
        sem_g = scr[3 * _NS + 1 + _NS:]
        wid = lax.axis_index("s") * 2 + lax.axis_index("c")
        base = wid * _ROWS_W

        def iw_issue(blk, s):
            roff = base + blk
            pltpu.async_copy(idx_hbm.at[pl.ds(roff * 128, 128)], idx_vs[s],
                             sem_iw[s])
            pltpu.async_copy(w_hbm.at[pl.ds(roff * 256, 256)], w_vs[s],
                             sem_iw[s])

        def iw_wait(blk, s):
            roff = base + blk
            pltpu.make_async_copy(idx_hbm.at[pl.ds(roff * 128, 128)],
                                  idx_vs[s], sem_iw[s]).wait()
            pltpu.make_async_copy(w_hbm.at[pl.ds(roff * 256, 256)],
                                  w_vs[s], sem_iw[s]).wait()

        def g_issue(s):
            pltpu.async_copy(table_hbm.at[idx_vs[s]], rows_vs[s], sem_g[s])

        def g_wait(s):
            pltpu.make_async_copy(table_hbm.at[idx_vs[s]], rows_vs[s],
                                  sem_g[s]).wait()

        def compute(s, oloc):
            r = rows_vs[s]
            wv = w_vs[s]

            @pl.loop(0, 8)
            def _(h):
                acc0 = jnp.zeros((16,), f32)
                acc1 = jnp.zeros((16,), f32)
                for pt in range(16):
                    li = h * 16 + pt
                    s0 = jnp.full((16,), li, jnp.int32)
                    wab = plsc.bitcast(plsc.load_gather(wv, [s0]),
                                       jnp.bfloat16)
                    wcd = plsc.bitcast(plsc.load_gather(wv, [s0 + 128]),
                                       jnp.bfloat16)
                    v00, v01 = plsc.unpack(wab,
                                           format=plsc.PackFormat.INTERLEAVED)
                    v10, v11 = plsc.unpack(wcd,
                                           format=plsc.PackFormat.INTERLEAVED)
                    acc0 = (acc0 + v00 * r[li, pl.ds(0, 16)]
                            + v01 * r[li, pl.ds(32, 16)]
                            + v10 * r[li, pl.ds(64, 16)]
                            + v11 * r[li, pl.ds(96, 16)])
                    acc1 = (acc1 + v00 * r[li, pl.ds(16, 16)]
                            + v01 * r[li, pl.ds(48, 16)]
                            + v10 * r[li, pl.ds(80, 16)]
                            + v11 * r[li, pl.ds(112, 16)])
                out_v[pl.ds(oloc + h * 32, 16)] = acc0
                out_v[pl.ds(oloc + h * 32 + 16, 16)] = acc1

        for s in range(_NS):
            iw_issue(s, s)
        for s in range(2):
            iw_wait(s, s)
            g_issue(s)

        @pl.loop(0, _NJ)
        def _(j):
            local = j - (j // _FLJ) * _FLJ
            oloc = local * (_NS * 256)
            for s in range(_NS):
                blk = _NS * j + s
                g_wait(s)
                compute(s, oloc + s * 256)

                @pl.when(blk + _NS < _ROWS_W)
                def _():
                    iw_issue(blk + _NS, s)

                nblk = _NS * j + s + 2
                ns = (s + 2) % _NS

                @pl.when(nblk < _ROWS_W)
                def _():
                    iw_wait(nblk, ns)
                    g_issue(ns)

            @pl.when(local == _FLJ - 1)
            def _():
                fl = j // _FLJ
                pltpu.sync_copy(
                    out_v,
                    out_hbm.at[pl.ds(base * 256 + fl * (_FLROWS * 256),
                                     _FLROWS * 256)])

    return k(table, idx_flat, w_flat)


def kernel(query, reference_points, input_flatten, input_spatial_shapes,
           input_level_start_index, W_off, b_off, W_attn, b_attn, W_val,
           b_val, W_out, b_out):
    f32 = jnp.float32
    q2 = query.reshape(_NROW, 256)
    x2 = input_flatten.reshape(_NROW, 256)
    rp = reference_points.reshape(_NROW, 4, 2)
    rx = rp[:, _L_OF, 0]
    ry = rp[:, _L_OF, 1]
    bf16 = jnp.bfloat16
    val, idx, w4 = _run_proj(
        q2, x2,
        W_val.T.astype(bf16), b_val.reshape(1, 256),
        W_off[0::2].T.astype(bf16), b_off[0::2].reshape(1, 128),
        W_off[1::2].T.astype(bf16), b_off[1::2].reshape(1, 128),
        W_attn.T.astype(bf16), b_attn.reshape(1, 128),
        rx, ry)
    table = _build_table(val)
    acc = _sc_sample(table, idx.reshape(-1), w4.reshape(-1))
    out = _run_matmul(acc.reshape(_NROW, 256), W_out.T.astype(bf16),
                      b_out.reshape(1, 256))
    return out.reshape(_B, _LQ, 256)
